# baseline (speedup 1.0000x reference)
import functools

import jax
import jax.numpy as jnp
from jax import lax
from jax.experimental import pallas as pl
from jax.experimental.pallas import tpu as pltpu
from jax.experimental.pallas import tpu_sc as plsc

N = 10000
E = 320000
D = 128
H = 8
DH = 16
ACC = 136

NC = 2
NS = 16
NW = NC * NS
EPW = E // NW
CHUNK = 40
NCHUNK = EPW // CHUNK
PAIRS = NCHUNK // 2 - 1
NPAD = 10240
RPT = NPAD // NS


def _edge_pass_body(tab_hbm, xw_hbm, src_hbm, dst_hbm, out_hbm,
                    srcv0, dstv0, rows0, sA0, sD0,
                    srcv1, dstv1, rows1, sA1, sD1,
                    msgA, msgB, sidxA, sidxB, acc,
                    semG0, semG1, semSA, semSB):
    c = lax.axis_index("c")
    s = lax.axis_index("s")
    wid = s * NC + c

    zeros16 = jnp.zeros((16,), jnp.float32)
    zeros16i = jnp.zeros((16,), jnp.int32)

    def zrow(r, carry):
        for k in range(ACC // 16):
            msgA[r, pl.ds(16 * k, 16)] = zeros16
            msgB[r, pl.ds(16 * k, 16)] = zeros16
        msgA[r, pl.ds(ACC - 16, 16)] = zeros16
        msgB[r, pl.ds(ACC - 16, 16)] = zeros16
        return carry

    lax.fori_loop(0, CHUNK, zrow, 0)
    for k in range(CHUNK // 16):
        sidxA[pl.ds(16 * k, 16)] = zeros16i
        sidxB[pl.ds(16 * k, 16)] = zeros16i
    sidxA[pl.ds(CHUNK - 16, 16)] = zeros16i
    sidxB[pl.ds(CHUNK - 16, 16)] = zeros16i
    base_row = s * RPT
    for k in range(RPT // CHUNK):
        pltpu.sync_copy(msgA, acc.at[pl.ds(base_row + CHUNK * k, CHUNK)])
    plsc.subcore_barrier()

    pltpu.async_copy(msgA, acc.at[sidxA], semSA, add=True)
    pltpu.async_copy(msgB, acc.at[sidxB], semSB, add=True)

    ebase = wid * EPW

    def issue(g, srcv, dstv, rows, sA, sD, sem):
        off = ebase + g * CHUNK
        pltpu.sync_copy(src_hbm.at[pl.ds(off, CHUNK)], srcv)
        pltpu.sync_copy(dst_hbm.at[pl.ds(off, CHUNK)], dstv)
        cpR = pltpu.async_copy(xw_hbm.at[srcv], rows, sem)
        cpA = pltpu.async_copy(tab_hbm.at[srcv], sA, sem)
        cpD = pltpu.async_copy(tab_hbm.at[dstv], sD, sem)
        return cpR, cpA, cpD

    def drain(srcv, dstv, rows, sA, sD, sem):
        pltpu.make_async_copy(xw_hbm.at[srcv], rows, sem).wait()
        pltpu.make_async_copy(tab_hbm.at[srcv], sA, sem).wait()
        pltpu.make_async_copy(tab_hbm.at[dstv], sD, sem).wait()

    def compute(rows, sA, sD, msg):
        def edge_body(i, ecarry):
            a16 = sA[i]
            d16 = sD[i]
            e16 = a16 + lax.rev(d16, (0,))
            e16 = jnp.maximum(e16, 0.2 * e16)
            ex = jnp.exp(e16)
            msg[i, pl.ds(0, 16)] = ex
            for h in range(H):
                msg[i, pl.ds(8 + 16 * h, 16)] = (
                    rows[i, pl.ds(16 * h, 16)] * ex[h])
            return ecarry

        lax.fori_loop(0, CHUNK, edge_body, 0)

    def wait_scatter(msg, sidx, sem):
        pltpu.make_async_copy(msg, acc.at[sidx], sem).wait()

    def scatter(msg, dstv, sidx, sem):
        for k in range(CHUNK // 16):
            sidx[pl.ds(16 * k, 16)] = dstv[pl.ds(16 * k, 16)]
        sidx[pl.ds(CHUNK - 16, 16)] = dstv[pl.ds(CHUNK - 16, 16)]
        pltpu.async_copy(msg, acc.at[sidx], sem, add=True)

    issue(0, srcv0, dstv0, rows0, sA0, sD0, semG0)

    def pair_body(p, carry):
        g0 = 2 * p
        drain(srcv0, dstv0, rows0, sA0, sD0, semG0)
        cps = issue(g0 + 1, srcv1, dstv1, rows1, sA1, sD1, semG1)
        wait_scatter(msgA, sidxA, semSA)
        compute(rows0, sA0, sD0, msgA)
        scatter(msgA, dstv0, sidxA, semSA)
        issue(g0 + 2, srcv0, dstv0, rows0, sA0, sD0, semG0)
        for cp in cps:
            cp.wait()
        wait_scatter(msgB, sidxB, semSB)
        compute(rows1, sA1, sD1, msgB)
        scatter(msgB, dstv1, sidxB, semSB)
        return carry

    lax.fori_loop(0, PAIRS, pair_body, 0)
    drain(srcv0, dstv0, rows0, sA0, sD0, semG0)
    cps = issue(NCHUNK - 1, srcv1, dstv1, rows1, sA1, sD1, semG1)
    wait_scatter(msgA, sidxA, semSA)
    compute(rows0, sA0, sD0, msgA)
    scatter(msgA, dstv0, sidxA, semSA)
    for cp in cps:
        cp.wait()
    wait_scatter(msgB, sidxB, semSB)
    compute(rows1, sA1, sD1, msgB)
    scatter(msgB, dstv1, sidxB, semSB)
    wait_scatter(msgA, sidxA, semSA)
    wait_scatter(msgB, sidxB, semSB)
    plsc.subcore_barrier()

    pltpu.sync_copy(acc.at[pl.ds(base_row, RPT)],
                    out_hbm.at[c, pl.ds(base_row, RPT)])


@functools.cache
def _edge_pass():
    return functools.partial(
        pl.kernel,
        out_type=jax.ShapeDtypeStruct((NC, NPAD, ACC), jnp.float32),
        mesh=plsc.VectorSubcoreMesh(core_axis_name="c", subcore_axis_name="s"),
        scratch_types=[
            pltpu.VMEM((CHUNK,), jnp.int32),
            pltpu.VMEM((CHUNK,), jnp.int32),
            pltpu.VMEM((CHUNK, D), jnp.float32),
            pltpu.VMEM((CHUNK, 16), jnp.float32),
            pltpu.VMEM((CHUNK, 16), jnp.float32),
            pltpu.VMEM((CHUNK,), jnp.int32),
            pltpu.VMEM((CHUNK,), jnp.int32),
            pltpu.VMEM((CHUNK, D), jnp.float32),
            pltpu.VMEM((CHUNK, 16), jnp.float32),
            pltpu.VMEM((CHUNK, 16), jnp.float32),
            pltpu.VMEM((CHUNK, ACC), jnp.float32),
            pltpu.VMEM((CHUNK, ACC), jnp.float32),
            pltpu.VMEM((CHUNK,), jnp.int32),
            pltpu.VMEM((CHUNK,), jnp.int32),
            pltpu.VMEM_SHARED((NPAD, ACC), jnp.float32),
            pltpu.SemaphoreType.DMA,
            pltpu.SemaphoreType.DMA,
            pltpu.SemaphoreType.DMA,
            pltpu.SemaphoreType.DMA,
        ],
        compiler_params=pltpu.CompilerParams(use_tc_tiling_on_sc=False),
    )(_edge_pass_body)


def _prep_body(x_ref, w_ref, ac_ref, xw_ref, tab_ref):
    xw = jnp.dot(x_ref[...], w_ref[...], preferred_element_type=jnp.float32)
    xw_ref[...] = xw
    tab_ref[...] = jnp.dot(xw, ac_ref[...], preferred_element_type=jnp.float32)


_prep = pl.pallas_call(
    _prep_body,
    out_shape=[jax.ShapeDtypeStruct((N, D), jnp.float32),
               jax.ShapeDtypeStruct((N, 16), jnp.float32)],
)


def _mid_body(a0_ref, a1_ref, p_ref, b_ref, w_ref, ac_ref, xw_ref, tab_ref):
    d = a0_ref[...][:N] + a1_ref[...][:N]
    den = jnp.dot(d[:, :H], p_ref[...],
                  preferred_element_type=jnp.float32) + 1e-16
    h1 = jnp.maximum(d[:, H:] / den + b_ref[...], 0.0)
    xw = jnp.dot(h1, w_ref[...], preferred_element_type=jnp.float32)
    xw_ref[...] = xw
    tab_ref[...] = jnp.dot(xw, ac_ref[...], preferred_element_type=jnp.float32)


_mid = pl.pallas_call(
    _mid_body,
    out_shape=[jax.ShapeDtypeStruct((N, D), jnp.float32),
               jax.ShapeDtypeStruct((N, 16), jnp.float32)],
)


def _final_body(a0_ref, a1_ref, p_ref, b_ref, out_ref):
    d = a0_ref[...][:N] + a1_ref[...][:N]
    den = jnp.dot(d[:, :H], p_ref[...],
                  preferred_element_type=jnp.float32) + 1e-16
    out_ref[...] = jnp.maximum(d[:, H:] / den + b_ref[...], 0.0)


_final = pl.pallas_call(
    _final_body,
    out_shape=jax.ShapeDtypeStruct((N, D), jnp.float32),
)


def _build_acomb(a_src, a_dst):
    A = jnp.zeros((D, 16), jnp.float32)
    for h in range(H):
        A = A.at[h * DH:(h + 1) * DH, h].set(a_src[h])
        A = A.at[h * DH:(h + 1) * DH, 15 - h].set(a_dst[h])
    return A


def _build_p8():
    P = jnp.zeros((H, D), jnp.float32)
    for h in range(H):
        P = P.at[h, h * DH:(h + 1) * DH].set(1.0)
    return P


def kernel(x, edge_index, W1, a_src1, a_dst1, b1, W2, a_src2, a_dst2, b2):
    src32 = edge_index[0].astype(jnp.int32)
    dst32 = edge_index[1].astype(jnp.int32)
    p8 = _build_p8()
    ac1 = _build_acomb(a_src1, a_dst1)
    ac2 = _build_acomb(a_src2, a_dst2)
    b1r = b1.reshape(1, D)
    b2r = b2.reshape(1, D)

    edge_pass = _edge_pass()
    xw1, tab1 = _prep(x, W1, ac1)
    acc1 = edge_pass(tab1, xw1, src32, dst32)
    xw2, tab2 = _mid(acc1[0], acc1[1], p8, b1r, W2, ac2)
    acc2 = edge_pass(tab2, xw2, src32, dst32)
    return _final(acc2[0], acc2[1], p8, b2r)


# speedup vs baseline: 1.2041x; 1.2041x over previous
import functools

import jax
import jax.numpy as jnp
from jax import lax
from jax.experimental import pallas as pl
from jax.experimental.pallas import tpu as pltpu
from jax.experimental.pallas import tpu_sc as plsc

N = 10000
E = 320000
D = 128
H = 8
DH = 16
ACC = 136

NC = 2
NS = 16
NW = NC * NS
EPW = E // NW
CHUNK = 80
NCHUNK = EPW // CHUNK
SUP = 25
NSUP = NCHUNK // SUP
SPAIRS = SUP // 2
NPAD = 10112
RPT = NPAD // NS


def _edge_pass_body(tab_hbm, xw_hbm, idx_hbm, out_hbm,
                    ibuf0, ibuf1,
                    rows0, sA0, sD0,
                    rows1, sA1, sD1,
                    msg, acc, semG0, semG1, semI):
    c = lax.axis_index("c")
    s = lax.axis_index("s")
    wid = s * NC + c

    zeros16 = jnp.zeros((16,), jnp.float32)

    def zrow(r, carry):
        for k in range(ACC // 16):
            msg[r, pl.ds(16 * k, 16)] = zeros16
        msg[r, pl.ds(ACC - 16, 16)] = zeros16
        return carry

    lax.fori_loop(0, CHUNK, zrow, 0)
    base_row = s * RPT
    for k in range(RPT // CHUNK):
        pltpu.sync_copy(msg, acc.at[pl.ds(base_row + CHUNK * k, CHUNK)])
    pltpu.sync_copy(msg, acc.at[pl.ds(base_row + RPT - CHUNK, CHUNK)])
    plsc.subcore_barrier()


    def issue(buf, k, rows, sA, sD, sem):
        cpR = pltpu.async_copy(xw_hbm.at[buf.at[k]], rows, sem)
        cpA = pltpu.async_copy(tab_hbm.at[buf.at[k]], sA, sem)
        cpD = pltpu.async_copy(tab_hbm.at[buf.at[SUP + k]], sD, sem)
        return cpR, cpA, cpD

    def drain(buf, k, rows, sA, sD, sem):
        pltpu.make_async_copy(xw_hbm.at[buf.at[k]], rows, sem).wait()
        pltpu.make_async_copy(tab_hbm.at[buf.at[k]], sA, sem).wait()
        pltpu.make_async_copy(tab_hbm.at[buf.at[SUP + k]], sD, sem).wait()

    def compute_scatter(buf, k, rows, sA, sD):
        def edge_body(i, ecarry):
            a16 = sA[i]
            d16 = sD[i]
            e16 = a16 + lax.rev(d16, (0,))
            e16 = jnp.maximum(e16, 0.2 * e16)
            ex = jnp.exp(e16)
            msg[i, pl.ds(0, 16)] = ex
            for h in range(H):
                msg[i, pl.ds(8 + 16 * h, 16)] = (
                    rows[i, pl.ds(16 * h, 16)] * ex[h])
            return ecarry

        lax.fori_loop(0, CHUNK, edge_body, 0)
        pltpu.sync_copy(msg, acc.at[buf.at[SUP + k]], add=True)

    ibufs = (ibuf0, ibuf1)
    rbase = wid * NSUP * 2 * SUP
    pltpu.sync_copy(idx_hbm.at[pl.ds(rbase, 2 * SUP)], ibuf0)

    for t in range(NSUP):
        buf = ibufs[t % 2]
        if t > 0:
            pltpu.make_async_copy(
                idx_hbm.at[pl.ds(rbase + t * 2 * SUP, 2 * SUP)], buf,
                semI).wait()
        if t < NSUP - 1:
            pltpu.async_copy(
                idx_hbm.at[pl.ds(rbase + (t + 1) * 2 * SUP, 2 * SUP)],
                ibufs[(t + 1) % 2], semI)

        issue(buf, 0, rows0, sA0, sD0, semG0)

        def pair_body(p, carry):
            g0 = 2 * p
            drain(buf, g0, rows0, sA0, sD0, semG0)
            cps = issue(buf, g0 + 1, rows1, sA1, sD1, semG1)
            compute_scatter(buf, g0, rows0, sA0, sD0)
            issue(buf, g0 + 2, rows0, sA0, sD0, semG0)
            for cp in cps:
                cp.wait()
            compute_scatter(buf, g0 + 1, rows1, sA1, sD1)
            return carry

        lax.fori_loop(0, SPAIRS, pair_body, 0)
        drain(buf, SUP - 1, rows0, sA0, sD0, semG0)
        compute_scatter(buf, SUP - 1, rows0, sA0, sD0)

    plsc.subcore_barrier()
    pltpu.sync_copy(acc.at[pl.ds(base_row, RPT)],
                    out_hbm.at[c, pl.ds(base_row, RPT)])


@functools.cache
def _edge_pass():
    return functools.partial(
        pl.kernel,
        out_type=jax.ShapeDtypeStruct((NC, NPAD, ACC), jnp.float32),
        mesh=plsc.VectorSubcoreMesh(core_axis_name="c", subcore_axis_name="s"),
        scratch_types=[
            pltpu.VMEM((2 * SUP, CHUNK), jnp.int32),
            pltpu.VMEM((2 * SUP, CHUNK), jnp.int32),
            pltpu.VMEM((CHUNK, D), jnp.float32),
            pltpu.VMEM((CHUNK, 16), jnp.float32),
            pltpu.VMEM((CHUNK, 16), jnp.float32),
            pltpu.VMEM((CHUNK, D), jnp.float32),
            pltpu.VMEM((CHUNK, 16), jnp.float32),
            pltpu.VMEM((CHUNK, 16), jnp.float32),
            pltpu.VMEM((CHUNK, ACC), jnp.float32),
            pltpu.VMEM_SHARED((NPAD, ACC), jnp.float32),
            pltpu.SemaphoreType.DMA,
            pltpu.SemaphoreType.DMA,
            pltpu.SemaphoreType.DMA,
        ],
        compiler_params=pltpu.CompilerParams(use_tc_tiling_on_sc=False),
    )(_edge_pass_body)


def _prep_body(x_ref, w_ref, ac_ref, xw_ref, tab_ref):
    xw = jnp.dot(x_ref[...], w_ref[...], preferred_element_type=jnp.float32)
    xw_ref[...] = xw
    tab_ref[...] = jnp.dot(xw, ac_ref[...], preferred_element_type=jnp.float32)


_prep = pl.pallas_call(
    _prep_body,
    out_shape=[jax.ShapeDtypeStruct((N, D), jnp.float32),
               jax.ShapeDtypeStruct((N, 16), jnp.float32)],
)


def _mid_body(a0_ref, a1_ref, p_ref, b_ref, w_ref, ac_ref, xw_ref, tab_ref):
    d = a0_ref[...][:N] + a1_ref[...][:N]
    den = jnp.dot(d[:, :H], p_ref[...],
                  preferred_element_type=jnp.float32) + 1e-16
    h1 = jnp.maximum(d[:, H:] / den + b_ref[...], 0.0)
    xw = jnp.dot(h1, w_ref[...], preferred_element_type=jnp.float32)
    xw_ref[...] = xw
    tab_ref[...] = jnp.dot(xw, ac_ref[...], preferred_element_type=jnp.float32)


_mid = pl.pallas_call(
    _mid_body,
    out_shape=[jax.ShapeDtypeStruct((N, D), jnp.float32),
               jax.ShapeDtypeStruct((N, 16), jnp.float32)],
)


def _final_body(a0_ref, a1_ref, p_ref, b_ref, out_ref):
    d = a0_ref[...][:N] + a1_ref[...][:N]
    den = jnp.dot(d[:, :H], p_ref[...],
                  preferred_element_type=jnp.float32) + 1e-16
    out_ref[...] = jnp.maximum(d[:, H:] / den + b_ref[...], 0.0)


_final = pl.pallas_call(
    _final_body,
    out_shape=jax.ShapeDtypeStruct((N, D), jnp.float32),
)


def _build_acomb(a_src, a_dst):
    A = jnp.zeros((D, 16), jnp.float32)
    for h in range(H):
        A = A.at[h * DH:(h + 1) * DH, h].set(a_src[h])
        A = A.at[h * DH:(h + 1) * DH, 15 - h].set(a_dst[h])
    return A


def _build_p8():
    P = jnp.zeros((H, D), jnp.float32)
    for h in range(H):
        P = P.at[h, h * DH:(h + 1) * DH].set(1.0)
    return P


def kernel(x, edge_index, W1, a_src1, a_dst1, b1, W2, a_src2, a_dst2, b2):
    src32 = edge_index[0].astype(jnp.int32)
    dst32 = edge_index[1].astype(jnp.int32)
    src_r = src32.reshape(NW, NSUP, SUP, CHUNK)
    dst_r = dst32.reshape(NW, NSUP, SUP, CHUNK)
    idx = jnp.concatenate([src_r, dst_r], axis=2).reshape(-1, CHUNK)
    p8 = _build_p8()
    ac1 = _build_acomb(a_src1, a_dst1)
    ac2 = _build_acomb(a_src2, a_dst2)
    b1r = b1.reshape(1, D)
    b2r = b2.reshape(1, D)

    edge_pass = _edge_pass()
    xw1, tab1 = _prep(x, W1, ac1)
    acc1 = edge_pass(tab1, xw1, idx)
    xw2, tab2 = _mid(acc1[0], acc1[1], p8, b1r, W2, ac2)
    acc2 = edge_pass(tab2, xw2, idx)
    return _final(acc2[0], acc2[1], p8, b2r)


# speedup vs baseline: 1.5232x; 1.2650x over previous
import functools

import jax
import jax.numpy as jnp
from jax import lax
from jax.experimental import pallas as pl
from jax.experimental.pallas import tpu as pltpu
from jax.experimental.pallas import tpu_sc as plsc

N = 10000
E = 320000
D = 128
H = 8
DH = 16
ACC = 136

NC = 2
NS = 16
NW = NC * NS
EPW = E // NW
CHUNK = 80
NCHUNK = EPW // CHUNK
SUP = 25
NSUP = NCHUNK // SUP
SPAIRS = SUP // 2
NPAD = 10112
RPT = NPAD // NS


def _edge_pass_body(tab_hbm, xw_hbm, idx_hbm, out_hbm,
                    ibuf0, ibuf1,
                    rows0, sA0, sD0,
                    rows1, sA1, sD1,
                    msg, acc, semG0, semG1, semI):
    c = lax.axis_index("c")
    s = lax.axis_index("s")
    wid = s * NC + c

    zeros16 = jnp.zeros((16,), jnp.float32)

    def zrow(r, carry):
        for k in range(ACC // 16):
            msg[r, pl.ds(16 * k, 16)] = zeros16
        msg[r, pl.ds(ACC - 16, 16)] = zeros16
        return carry

    lax.fori_loop(0, CHUNK, zrow, 0)
    base_row = s * RPT
    for k in range(RPT // CHUNK):
        pltpu.sync_copy(msg, acc.at[pl.ds(base_row + CHUNK * k, CHUNK)])
    pltpu.sync_copy(msg, acc.at[pl.ds(base_row + RPT - CHUNK, CHUNK)])
    plsc.subcore_barrier()


    def issue(buf, k, rows, sA, sD, sem):
        cpR = pltpu.async_copy(xw_hbm.at[buf.at[k]], rows, sem)
        cpA = pltpu.async_copy(tab_hbm.at[buf.at[k]], sA, sem)
        cpD = pltpu.async_copy(tab_hbm.at[buf.at[SUP + k]], sD, sem)
        return cpR, cpA, cpD

    def drain(buf, k, rows, sA, sD, sem):
        pltpu.make_async_copy(xw_hbm.at[buf.at[k]], rows, sem).wait()
        pltpu.make_async_copy(tab_hbm.at[buf.at[k]], sA, sem).wait()
        pltpu.make_async_copy(tab_hbm.at[buf.at[SUP + k]], sD, sem).wait()

    def compute_scatter(buf, k, rows, sA, sD):
        def edge_body(i, ecarry):
            a16 = sA[i]
            d16 = sD[i]
            e16 = a16 + lax.rev(d16, (0,))
            e16 = jnp.maximum(e16, 0.2 * e16)
            ex = jnp.exp(e16)
            msg[i, pl.ds(0, 16)] = ex
            for h in range(H // 2):
                v = rows[i, pl.ds(16 * h, 16)]
                lo = lax.bitcast_convert_type(
                    lax.shift_left(v, jnp.int32(16)), jnp.float32)
                hi = lax.bitcast_convert_type(
                    jnp.bitwise_and(v, jnp.int32(-65536)), jnp.float32)
                msg[i, pl.ds(8 + 16 * h, 16)] = lo * ex[h]
                msg[i, pl.ds(8 + 16 * (h + 4), 16)] = hi * ex[h + 4]
            return ecarry

        lax.fori_loop(0, CHUNK, edge_body, 0)
        pltpu.sync_copy(msg, acc.at[buf.at[SUP + k]], add=True)

    ibufs = (ibuf0, ibuf1)
    rbase = wid * NSUP * 2 * SUP
    pltpu.sync_copy(idx_hbm.at[pl.ds(rbase, 2 * SUP)], ibuf0)

    for t in range(NSUP):
        buf = ibufs[t % 2]
        if t > 0:
            pltpu.make_async_copy(
                idx_hbm.at[pl.ds(rbase + t * 2 * SUP, 2 * SUP)], buf,
                semI).wait()
        if t < NSUP - 1:
            pltpu.async_copy(
                idx_hbm.at[pl.ds(rbase + (t + 1) * 2 * SUP, 2 * SUP)],
                ibufs[(t + 1) % 2], semI)

        issue(buf, 0, rows0, sA0, sD0, semG0)

        def pair_body(p, carry):
            g0 = 2 * p
            drain(buf, g0, rows0, sA0, sD0, semG0)
            cps = issue(buf, g0 + 1, rows1, sA1, sD1, semG1)
            compute_scatter(buf, g0, rows0, sA0, sD0)
            issue(buf, g0 + 2, rows0, sA0, sD0, semG0)
            for cp in cps:
                cp.wait()
            compute_scatter(buf, g0 + 1, rows1, sA1, sD1)
            return carry

        lax.fori_loop(0, SPAIRS, pair_body, 0)
        drain(buf, SUP - 1, rows0, sA0, sD0, semG0)
        compute_scatter(buf, SUP - 1, rows0, sA0, sD0)

    plsc.subcore_barrier()
    pltpu.sync_copy(acc.at[pl.ds(base_row, RPT)],
                    out_hbm.at[c, pl.ds(base_row, RPT)])


@functools.cache
def _edge_pass():
    return functools.partial(
        pl.kernel,
        out_type=jax.ShapeDtypeStruct((NC, NPAD, ACC), jnp.float32),
        mesh=plsc.VectorSubcoreMesh(core_axis_name="c", subcore_axis_name="s"),
        scratch_types=[
            pltpu.VMEM((2 * SUP, CHUNK), jnp.int32),
            pltpu.VMEM((2 * SUP, CHUNK), jnp.int32),
            pltpu.VMEM((CHUNK, D // 2), jnp.int32),
            pltpu.VMEM((CHUNK, 16), jnp.float32),
            pltpu.VMEM((CHUNK, 16), jnp.float32),
            pltpu.VMEM((CHUNK, D // 2), jnp.int32),
            pltpu.VMEM((CHUNK, 16), jnp.float32),
            pltpu.VMEM((CHUNK, 16), jnp.float32),
            pltpu.VMEM((CHUNK, ACC), jnp.float32),
            pltpu.VMEM_SHARED((NPAD, ACC), jnp.float32),
            pltpu.SemaphoreType.DMA,
            pltpu.SemaphoreType.DMA,
            pltpu.SemaphoreType.DMA,
        ],
        compiler_params=pltpu.CompilerParams(use_tc_tiling_on_sc=False),
    )(_edge_pass_body)


def _pack_bf16(xw):
    u = lax.bitcast_convert_type(xw, jnp.uint32)
    lsb = jnp.bitwise_and(lax.shift_right_logical(u, jnp.uint32(16)),
                          jnp.uint32(1))
    ur = u + lsb + jnp.uint32(0x7FFF)
    hi = jnp.bitwise_and(ur, jnp.uint32(0xFFFF0000))
    packed = jnp.bitwise_or(
        lax.shift_right_logical(hi[:, :D // 2], jnp.uint32(16)),
        hi[:, D // 2:])
    return lax.bitcast_convert_type(packed, jnp.int32)


def _prep_body(x_ref, w_ref, ac_ref, xw_ref, tab_ref):
    xw = jnp.dot(x_ref[...], w_ref[...], preferred_element_type=jnp.float32)
    xw_ref[...] = _pack_bf16(xw)
    tab_ref[...] = jnp.dot(xw, ac_ref[...], preferred_element_type=jnp.float32)


_prep = pl.pallas_call(
    _prep_body,
    out_shape=[jax.ShapeDtypeStruct((N, D // 2), jnp.int32),
               jax.ShapeDtypeStruct((N, 16), jnp.float32)],
)


def _mid_body(a0_ref, a1_ref, p_ref, b_ref, w_ref, ac_ref, xw_ref, tab_ref):
    d = a0_ref[...][:N] + a1_ref[...][:N]
    den = jnp.dot(d[:, :H], p_ref[...],
                  preferred_element_type=jnp.float32) + 1e-16
    h1 = jnp.maximum(d[:, H:] / den + b_ref[...], 0.0)
    xw = jnp.dot(h1, w_ref[...], preferred_element_type=jnp.float32)
    xw_ref[...] = _pack_bf16(xw)
    tab_ref[...] = jnp.dot(xw, ac_ref[...], preferred_element_type=jnp.float32)


_mid = pl.pallas_call(
    _mid_body,
    out_shape=[jax.ShapeDtypeStruct((N, D // 2), jnp.int32),
               jax.ShapeDtypeStruct((N, 16), jnp.float32)],
)


def _final_body(a0_ref, a1_ref, p_ref, b_ref, out_ref):
    d = a0_ref[...][:N] + a1_ref[...][:N]
    den = jnp.dot(d[:, :H], p_ref[...],
                  preferred_element_type=jnp.float32) + 1e-16
    out_ref[...] = jnp.maximum(d[:, H:] / den + b_ref[...], 0.0)


_final = pl.pallas_call(
    _final_body,
    out_shape=jax.ShapeDtypeStruct((N, D), jnp.float32),
)


def _build_acomb(a_src, a_dst):
    A = jnp.zeros((D, 16), jnp.float32)
    for h in range(H):
        A = A.at[h * DH:(h + 1) * DH, h].set(a_src[h])
        A = A.at[h * DH:(h + 1) * DH, 15 - h].set(a_dst[h])
    return A


def _build_p8():
    P = jnp.zeros((H, D), jnp.float32)
    for h in range(H):
        P = P.at[h, h * DH:(h + 1) * DH].set(1.0)
    return P


def kernel(x, edge_index, W1, a_src1, a_dst1, b1, W2, a_src2, a_dst2, b2):
    src32 = edge_index[0].astype(jnp.int32)
    dst32 = edge_index[1].astype(jnp.int32)
    src_r = src32.reshape(NW, NSUP, SUP, CHUNK)
    dst_r = dst32.reshape(NW, NSUP, SUP, CHUNK)
    idx = jnp.concatenate([src_r, dst_r], axis=2).reshape(-1, CHUNK)
    p8 = _build_p8()
    ac1 = _build_acomb(a_src1, a_dst1)
    ac2 = _build_acomb(a_src2, a_dst2)
    b1r = b1.reshape(1, D)
    b2r = b2.reshape(1, D)

    edge_pass = _edge_pass()
    xw1, tab1 = _prep(x, W1, ac1)
    acc1 = edge_pass(tab1, xw1, idx)
    xw2, tab2 = _mid(acc1[0], acc1[1], p8, b1r, W2, ac2)
    acc2 = edge_pass(tab2, xw2, idx)
    return _final(acc2[0], acc2[1], p8, b2r)


# speedup vs baseline: 3.3458x; 2.1966x over previous
import functools

import jax
import jax.numpy as jnp
from jax import lax
from jax.experimental import pallas as pl
from jax.experimental.pallas import tpu as pltpu
from jax.experimental.pallas import tpu_sc as plsc

N = 10000
E = 320000
D = 128
H = 8
DH = 16
ACC = 136

NC = 2
NS = 16
NW = NC * NS
EPW = E // NW
CHUNK = 80
NCHUNK = EPW // CHUNK
SUP = 25
NSUP = NCHUNK // SUP
SPAIRS = SUP // 2
NPAD = 10112
RPT = NPAD // NS


def _edge_pass_body(tab_hbm, xw_hbm, idx_hbm, out_hbm,
                    ibuf0, ibuf1,
                    rows0, sA0, sD0,
                    rows1, sA1, sD1,
                    msg, acc, semG0, semG1, semI):
    c = lax.axis_index("c")
    s = lax.axis_index("s")
    wid = s * NC + c

    zeros16 = jnp.zeros((16,), jnp.float32)

    def zrow(r, carry):
        for k in range(ACC // 16):
            msg[r, pl.ds(16 * k, 16)] = zeros16
        msg[r, pl.ds(ACC - 16, 16)] = zeros16
        return carry

    lax.fori_loop(0, CHUNK, zrow, 0)
    base_row = s * RPT
    for k in range(RPT // CHUNK):
        pltpu.sync_copy(msg, acc.at[pl.ds(base_row + CHUNK * k, CHUNK)])
    pltpu.sync_copy(msg, acc.at[pl.ds(base_row + RPT - CHUNK, CHUNK)])
    plsc.subcore_barrier()


    def issue(buf, k, rows, sA, sD, sem):
        cpR = pltpu.async_copy(xw_hbm.at[buf.at[k]], rows, sem)
        cpA = pltpu.async_copy(tab_hbm.at[buf.at[k]], sA, sem)
        cpD = pltpu.async_copy(tab_hbm.at[buf.at[SUP + k]], sD, sem)
        return cpR, cpA, cpD

    def drain(buf, k, rows, sA, sD, sem):
        pltpu.make_async_copy(xw_hbm.at[buf.at[k]], rows, sem).wait()
        pltpu.make_async_copy(tab_hbm.at[buf.at[k]], sA, sem).wait()
        pltpu.make_async_copy(tab_hbm.at[buf.at[SUP + k]], sD, sem).wait()

    def compute_scatter(buf, k, rows, sA, sD):
        @plsc.parallel_loop(0, CHUNK, unroll=4)
        def edge_body(i):
            a16 = sA[i]
            d16 = sD[i]
            e16 = a16 + lax.rev(d16, (0,))
            e16 = jnp.maximum(e16, 0.2 * e16)
            ex = jnp.exp(e16)
            msg[i, pl.ds(0, 16)] = ex
            for h in range(H // 2):
                v = rows[i, pl.ds(16 * h, 16)]
                lo = lax.bitcast_convert_type(
                    lax.shift_left(v, jnp.int32(16)), jnp.float32)
                hi = lax.bitcast_convert_type(
                    jnp.bitwise_and(v, jnp.int32(-65536)), jnp.float32)
                msg[i, pl.ds(8 + 16 * h, 16)] = lo * ex[h]
                msg[i, pl.ds(8 + 16 * (h + 4), 16)] = hi * ex[h + 4]

        pltpu.sync_copy(msg, acc.at[buf.at[SUP + k]], add=True)

    ibufs = (ibuf0, ibuf1)
    rbase = wid * NSUP * 2 * SUP
    pltpu.sync_copy(idx_hbm.at[pl.ds(rbase, 2 * SUP)], ibuf0)

    for t in range(NSUP):
        buf = ibufs[t % 2]
        if t > 0:
            pltpu.make_async_copy(
                idx_hbm.at[pl.ds(rbase + t * 2 * SUP, 2 * SUP)], buf,
                semI).wait()
        if t < NSUP - 1:
            pltpu.async_copy(
                idx_hbm.at[pl.ds(rbase + (t + 1) * 2 * SUP, 2 * SUP)],
                ibufs[(t + 1) % 2], semI)

        issue(buf, 0, rows0, sA0, sD0, semG0)

        def pair_body(p, carry):
            g0 = 2 * p
            drain(buf, g0, rows0, sA0, sD0, semG0)
            cps = issue(buf, g0 + 1, rows1, sA1, sD1, semG1)
            compute_scatter(buf, g0, rows0, sA0, sD0)
            issue(buf, g0 + 2, rows0, sA0, sD0, semG0)
            for cp in cps:
                cp.wait()
            compute_scatter(buf, g0 + 1, rows1, sA1, sD1)
            return carry

        lax.fori_loop(0, SPAIRS, pair_body, 0)
        drain(buf, SUP - 1, rows0, sA0, sD0, semG0)
        compute_scatter(buf, SUP - 1, rows0, sA0, sD0)

    plsc.subcore_barrier()
    pltpu.sync_copy(acc.at[pl.ds(base_row, RPT)],
                    out_hbm.at[c, pl.ds(base_row, RPT)])


@functools.cache
def _edge_pass():
    return functools.partial(
        pl.kernel,
        out_type=jax.ShapeDtypeStruct((NC, NPAD, ACC), jnp.float32),
        mesh=plsc.VectorSubcoreMesh(core_axis_name="c", subcore_axis_name="s"),
        scratch_types=[
            pltpu.VMEM((2 * SUP, CHUNK), jnp.int32),
            pltpu.VMEM((2 * SUP, CHUNK), jnp.int32),
            pltpu.VMEM((CHUNK, D // 2), jnp.int32),
            pltpu.VMEM((CHUNK, 16), jnp.float32),
            pltpu.VMEM((CHUNK, 16), jnp.float32),
            pltpu.VMEM((CHUNK, D // 2), jnp.int32),
            pltpu.VMEM((CHUNK, 16), jnp.float32),
            pltpu.VMEM((CHUNK, 16), jnp.float32),
            pltpu.VMEM((CHUNK, ACC), jnp.float32),
            pltpu.VMEM_SHARED((NPAD, ACC), jnp.float32),
            pltpu.SemaphoreType.DMA,
            pltpu.SemaphoreType.DMA,
            pltpu.SemaphoreType.DMA,
        ],
        compiler_params=pltpu.CompilerParams(use_tc_tiling_on_sc=False),
    )(_edge_pass_body)


def _pack_bf16(xw):
    u = lax.bitcast_convert_type(xw, jnp.uint32)
    lsb = jnp.bitwise_and(lax.shift_right_logical(u, jnp.uint32(16)),
                          jnp.uint32(1))
    ur = u + lsb + jnp.uint32(0x7FFF)
    hi = jnp.bitwise_and(ur, jnp.uint32(0xFFFF0000))
    packed = jnp.bitwise_or(
        lax.shift_right_logical(hi[:, :D // 2], jnp.uint32(16)),
        hi[:, D // 2:])
    return lax.bitcast_convert_type(packed, jnp.int32)


def _prep_body(x_ref, w_ref, ac_ref, xw_ref, tab_ref):
    xw = jnp.dot(x_ref[...], w_ref[...], preferred_element_type=jnp.float32)
    xw_ref[...] = _pack_bf16(xw)
    tab_ref[...] = jnp.dot(xw, ac_ref[...], preferred_element_type=jnp.float32)


_prep = pl.pallas_call(
    _prep_body,
    out_shape=[jax.ShapeDtypeStruct((N, D // 2), jnp.int32),
               jax.ShapeDtypeStruct((N, 16), jnp.float32)],
)


def _mid_body(a0_ref, a1_ref, p_ref, b_ref, w_ref, ac_ref, xw_ref, tab_ref):
    d = a0_ref[...][:N] + a1_ref[...][:N]
    den = jnp.dot(d[:, :H], p_ref[...],
                  preferred_element_type=jnp.float32) + 1e-16
    h1 = jnp.maximum(d[:, H:] / den + b_ref[...], 0.0)
    xw = jnp.dot(h1, w_ref[...], preferred_element_type=jnp.float32)
    xw_ref[...] = _pack_bf16(xw)
    tab_ref[...] = jnp.dot(xw, ac_ref[...], preferred_element_type=jnp.float32)


_mid = pl.pallas_call(
    _mid_body,
    out_shape=[jax.ShapeDtypeStruct((N, D // 2), jnp.int32),
               jax.ShapeDtypeStruct((N, 16), jnp.float32)],
)


def _final_body(a0_ref, a1_ref, p_ref, b_ref, out_ref):
    d = a0_ref[...][:N] + a1_ref[...][:N]
    den = jnp.dot(d[:, :H], p_ref[...],
                  preferred_element_type=jnp.float32) + 1e-16
    out_ref[...] = jnp.maximum(d[:, H:] / den + b_ref[...], 0.0)


_final = pl.pallas_call(
    _final_body,
    out_shape=jax.ShapeDtypeStruct((N, D), jnp.float32),
)


def _build_acomb(a_src, a_dst):
    A = jnp.zeros((D, 16), jnp.float32)
    for h in range(H):
        A = A.at[h * DH:(h + 1) * DH, h].set(a_src[h])
        A = A.at[h * DH:(h + 1) * DH, 15 - h].set(a_dst[h])
    return A


def _build_p8():
    P = jnp.zeros((H, D), jnp.float32)
    for h in range(H):
        P = P.at[h, h * DH:(h + 1) * DH].set(1.0)
    return P


def kernel(x, edge_index, W1, a_src1, a_dst1, b1, W2, a_src2, a_dst2, b2):
    src32 = edge_index[0].astype(jnp.int32)
    dst32 = edge_index[1].astype(jnp.int32)
    src_r = src32.reshape(NW, NSUP, SUP, CHUNK)
    dst_r = dst32.reshape(NW, NSUP, SUP, CHUNK)
    idx = jnp.concatenate([src_r, dst_r], axis=2).reshape(-1, CHUNK)
    p8 = _build_p8()
    ac1 = _build_acomb(a_src1, a_dst1)
    ac2 = _build_acomb(a_src2, a_dst2)
    b1r = b1.reshape(1, D)
    b2r = b2.reshape(1, D)

    edge_pass = _edge_pass()
    xw1, tab1 = _prep(x, W1, ac1)
    acc1 = edge_pass(tab1, xw1, idx)
    xw2, tab2 = _mid(acc1[0], acc1[1], p8, b1r, W2, ac2)
    acc2 = edge_pass(tab2, xw2, idx)
    return _final(acc2[0], acc2[1], p8, b2r)
